# baseline (speedup 1.0000x reference)
import jax
import jax.numpy as jnp
from jax.experimental import pallas as pl

IN_CHANNELS = 256
HIDDEN_DIM = 128
NUM_EXPERTS = 4
NSPLIT = 4


def _gate_kernel(x0, x1, x2, x3, w1_ref, b1_ref, w2_ref, b2_ref, out_ref):
    b = pl.program_id(0)
    cols = []
    for xr in (x0, x1, x2, x3):
        x = xr[0]
        s1 = jnp.sum(x, axis=1)
        cols.append(jnp.sum(s1, axis=1, keepdims=True))
    hw = x0.shape[2] * x0.shape[3]
    pooled = jnp.concatenate(cols, axis=0) * (1.0 / hw)
    h = jax.lax.dot_general(
        pooled, w1_ref[...], (((0,), (0,)), ((), ())),
        preferred_element_type=jnp.float32)
    h = h + b1_ref[...]
    h = 0.5 * h * (1.0 + jax.lax.erf(h * 0.7071067811865476))
    logits = jnp.dot(h, w2_ref[...], preferred_element_type=jnp.float32)
    logits = logits + b2_ref[...]
    m = jnp.max(logits, axis=-1, keepdims=True)
    e = jnp.exp(logits - m)
    weights = e / jnp.sum(e, axis=-1, keepdims=True)
    out_ref[pl.ds(b, 1), :] = weights


@jax.jit
def kernel(img_emb, W1, b1, W2, b2):
    B, C, H, W = img_emb.shape
    Cq = C // NSPLIT
    b1r = b1.reshape(1, HIDDEN_DIM)
    b2r = b2.reshape(1, NUM_EXPERTS)

    def qspec(k):
        return pl.BlockSpec((1, Cq, H, W), lambda b, k=k: (b, k, 0, 0))

    out = pl.pallas_call(
        _gate_kernel,
        grid=(B,),
        in_specs=[qspec(0), qspec(1), qspec(2), qspec(3),
                  pl.BlockSpec((C, HIDDEN_DIM), lambda b: (0, 0)),
                  pl.BlockSpec((1, HIDDEN_DIM), lambda b: (0, 0)),
                  pl.BlockSpec((HIDDEN_DIM, NUM_EXPERTS), lambda b: (0, 0)),
                  pl.BlockSpec((1, NUM_EXPERTS), lambda b: (0, 0))],
        out_specs=pl.BlockSpec((B, NUM_EXPERTS), lambda b: (0, 0)),
        out_shape=jax.ShapeDtypeStruct((B, NUM_EXPERTS), jnp.float32),
    )(img_emb, img_emb, img_emb, img_emb, W1, b1r, W2, b2r)
    return out


# speedup vs baseline: 5.3980x; 5.3980x over previous
import jax
import jax.numpy as jnp
from jax.experimental import pallas as pl

IN_CHANNELS = 256
HIDDEN_DIM = 128
NUM_EXPERTS = 4


def _gate_kernel(x_ref, w1_ref, b1_ref, w2_ref, b2_ref, out_ref):
    b = pl.program_id(0)
    x = x_ref[0]
    hw = x.shape[0] * x.shape[1]
    pooled = jnp.sum(x, axis=(0, 1)) * (1.0 / hw)
    pooled = pooled.reshape(1, -1)
    h = jnp.dot(pooled, w1_ref[...], preferred_element_type=jnp.float32)
    h = h + b1_ref[...]
    h = 0.5 * h * (1.0 + jax.lax.erf(h * 0.7071067811865476))
    logits = jnp.dot(h, w2_ref[...], preferred_element_type=jnp.float32)
    logits = logits + b2_ref[...]
    m = jnp.max(logits, axis=-1, keepdims=True)
    e = jnp.exp(logits - m)
    weights = e / jnp.sum(e, axis=-1, keepdims=True)
    out_ref[pl.ds(b, 1), :] = weights


@jax.jit
def kernel(img_emb, W1, b1, W2, b2):
    B, C, H, W = img_emb.shape
    x = img_emb.transpose(0, 2, 3, 1)
    b1r = b1.reshape(1, HIDDEN_DIM)
    b2r = b2.reshape(1, NUM_EXPERTS)
    out = pl.pallas_call(
        _gate_kernel,
        grid=(B,),
        in_specs=[
            pl.BlockSpec((1, H, W, C), lambda b: (b, 0, 0, 0)),
            pl.BlockSpec((C, HIDDEN_DIM), lambda b: (0, 0)),
            pl.BlockSpec((1, HIDDEN_DIM), lambda b: (0, 0)),
            pl.BlockSpec((HIDDEN_DIM, NUM_EXPERTS), lambda b: (0, 0)),
            pl.BlockSpec((1, NUM_EXPERTS), lambda b: (0, 0)),
        ],
        out_specs=pl.BlockSpec((B, NUM_EXPERTS), lambda b: (0, 0)),
        out_shape=jax.ShapeDtypeStruct((B, NUM_EXPERTS), jnp.float32),
    )(x, W1, b1r, W2, b2r)
    return out


# speedup vs baseline: 6.0038x; 1.1122x over previous
import jax
import jax.numpy as jnp
from jax.experimental import pallas as pl

IN_CHANNELS = 256
HIDDEN_DIM = 128
NUM_EXPERTS = 4


def _gate_kernel(x_ref, w1_ref, b1_ref, w2_ref, b2_ref, out_ref):
    b = pl.program_id(0)
    x = x_ref[0]
    hw = x.shape[0] * x.shape[1]
    part = jnp.sum(x, axis=0)
    pooled = jnp.sum(part, axis=0) * (1.0 / hw)
    pooled = pooled.reshape(1, -1)
    h = jnp.dot(pooled, w1_ref[...], preferred_element_type=jnp.float32)
    h = h + b1_ref[...]
    h = 0.5 * h * (1.0 + jax.lax.erf(h * 0.7071067811865476))
    logits = jnp.dot(h, w2_ref[...], preferred_element_type=jnp.float32)
    logits = logits + b2_ref[...]
    m = jnp.max(logits, axis=-1, keepdims=True)
    e = jnp.exp(logits - m)
    weights = e / jnp.sum(e, axis=-1, keepdims=True)
    out_ref[pl.ds(b, 1), :] = weights


@jax.jit
def kernel(img_emb, W1, b1, W2, b2):
    B, C, H, W = img_emb.shape
    x = img_emb.transpose(0, 2, 3, 1)
    b1r = b1.reshape(1, HIDDEN_DIM)
    b2r = b2.reshape(1, NUM_EXPERTS)
    out = pl.pallas_call(
        _gate_kernel,
        grid=(B,),
        in_specs=[
            pl.BlockSpec((1, H, W, C), lambda b: (b, 0, 0, 0)),
            pl.BlockSpec((C, HIDDEN_DIM), lambda b: (0, 0)),
            pl.BlockSpec((1, HIDDEN_DIM), lambda b: (0, 0)),
            pl.BlockSpec((HIDDEN_DIM, NUM_EXPERTS), lambda b: (0, 0)),
            pl.BlockSpec((1, NUM_EXPERTS), lambda b: (0, 0)),
        ],
        out_specs=pl.BlockSpec((B, NUM_EXPERTS), lambda b: (0, 0)),
        out_shape=jax.ShapeDtypeStruct((B, NUM_EXPERTS), jnp.float32),
    )(x, W1, b1r, W2, b2r)
    return out
